# Initial kernel scaffold; baseline (speedup 1.0000x reference)
import jax
import jax.numpy as jnp
from jax.experimental import pallas as pl


def kernel(ts, ts_w, pos_w):
    raise NotImplementedError("write your pallas kernel here")


# speedup vs baseline: 1055.1077x; 1055.1077x over previous
import jax
import jax.numpy as jnp
from jax.experimental import pallas as pl
from jax.experimental.pallas import tpu as pltpu

_N = 200
_NB = 64
_BB = 8


def _pb_kernel(posw_ref, out_ref):
    for i in range(_N):
        out_ref[i, :] = posw_ref[0, _N - 1 - i : 2 * _N - 1 - i]


def _main_kernel(ts_ref, tsw_ref, pb_ref, out_ref):
    ts = ts_ref[...]
    shifted = jnp.concatenate([ts[:, 1:], ts[:, _N - 1 : _N]], axis=1)
    diff = shifted[:, :, None] - ts[:, None, :]
    mag = jnp.maximum(jnp.abs(diff), 1).astype(jnp.float32)
    bk = jnp.clip((jnp.log(mag) / 0.301).astype(jnp.int32), 0, _NB)
    bk2 = bk.reshape(_BB * _N, _N)
    table = jnp.broadcast_to(tsw_ref[0, :128], (_BB * _N, 128))
    g = jnp.take_along_axis(table, bk2, axis=-1).reshape(_BB, _N, _N)
    out_ref[...] = g + pb_ref[...][None, :, :]


def kernel(ts, ts_w, pos_w):
    B, N = ts.shape
    posw_pad = jnp.zeros((1, 512), jnp.float32).at[0, : 2 * N - 1].set(pos_w)
    pb = pl.pallas_call(
        _pb_kernel,
        out_shape=jax.ShapeDtypeStruct((N, N), jnp.float32),
    )(posw_pad)

    tsw_pad = jnp.zeros((1, 256), jnp.float32).at[0, : _NB + 1].set(ts_w)
    return pl.pallas_call(
        _main_kernel,
        grid=(B // _BB,),
        in_specs=[
            pl.BlockSpec((_BB, N), lambda b: (b, 0)),
            pl.BlockSpec((1, 256), lambda b: (0, 0)),
            pl.BlockSpec((N, N), lambda b: (0, 0)),
        ],
        out_specs=pl.BlockSpec((_BB, N, N), lambda b: (b, 0, 0)),
        out_shape=jax.ShapeDtypeStruct((B, N, N), jnp.float32),
        compiler_params=pltpu.CompilerParams(
            dimension_semantics=("arbitrary",),
        ),
    )(ts, tsw_pad, pb)
